# Initial kernel scaffold; baseline (speedup 1.0000x reference)
import jax
import jax.numpy as jnp
from jax.experimental import pallas as pl


def kernel(e_j, e_s, g_j0, g_s0, adj_val, Wj, bj, Ws, bs, Wja, bja, Wsa, bsa, adj_row, adj_col, j_ids, s_ids, negs):
    raise NotImplementedError("write your pallas kernel here")


# speedup vs baseline: 1.0003x; 1.0003x over previous
import jax
import jax.numpy as jnp
from jax.experimental import pallas as pl


def _normalize(x, eps=1e-12):
    n = jnp.linalg.norm(x, axis=1, keepdims=True)
    return x / jnp.maximum(n, eps)


def _combine_kernel(a_ref, b_ref, o_ref):
    o_ref[...] = a_ref[...] + b_ref[...]


def kernel(e_j, e_s, g_j0, g_s0, adj_val, Wj, bj, Ws, bs, Wja, bja, Wsa, bsa,
           adj_row, adj_col, j_ids, s_ids, negs):
    nj, ns = e_j.shape[0], e_s.shape[0]
    nl = Wj.shape[0]
    act = jax.nn.leaky_relu
    TEMP, LAMBDA_1 = 0.2, 1e-4

    def spmm(X):
        return jax.ops.segment_sum(adj_val[:, None] * X[adj_col], adj_row, num_segments=nj)

    def spmm_t(X):
        return jax.ops.segment_sum(adj_val[:, None] * X[adj_row], adj_col, num_segments=ns)

    Ej = [e_j]
    Es = [e_s]
    Gj = [_normalize(g_j0)]
    Gs = [_normalize(g_s0)]
    for l in range(nl):
        Ej.append(Ej[l] + act(spmm(Es[l]) @ Wj[l].T + bj[l]))
        Es.append(Es[l] + act(spmm_t(Ej[l]) @ Ws[l].T + bs[l]))
        Gj.append(Gj[l] + act(spmm(Gs[l]) @ Wja[l].T + bja[l]))
        Gs.append(Gs[l] + act(spmm_t(Gj[l]) @ Wsa[l].T + bsa[l]))

    E_j = _normalize(sum(Ej) / len(Ej))
    E_s = _normalize(sum(Es) / len(Es))
    G_j = _normalize(sum(Gj) / len(Gj))
    G_s = _normalize(sum(Gs) / len(Gs))

    neg_emb = E_s[negs]
    neg_emb_T = jnp.transpose(neg_emb, (1, 2, 0))

    neg_score = jnp.log(jnp.exp(G_j[j_ids] @ E_j[j_ids].T / TEMP).sum(1) + 1e-08).mean()
    sc = jnp.einsum('bd,cdn->bcn', G_s[s_ids], neg_emb_T)
    neg_score = neg_score + jnp.log(jnp.exp(sc / TEMP).sum(1) + 1e-08).mean()

    pos_score = jnp.clip((G_j[j_ids] * E_j[j_ids]).sum(1) / TEMP, -1.0, 1.0).mean() \
        + jnp.clip((G_s[s_ids] * E_s[s_ids]).sum(1) / TEMP, -1.0, 1.0).mean()

    loss_cl = (-pos_score + neg_score) * 0.2
    params = [Wj, bj, Ws, bs, Wja, bja, Wsa, bsa]
    loss_reg = sum(jnp.sum(p * p) for p in params) * LAMBDA_1

    loss = pl.pallas_call(
        _combine_kernel,
        out_shape=jax.ShapeDtypeStruct((1, 1), jnp.float32),
    )(loss_cl.reshape(1, 1), loss_reg.reshape(1, 1))[0, 0]
    return (loss, loss_cl, loss_reg)


# speedup vs baseline: 2.9328x; 2.9321x over previous
import jax
import jax.numpy as jnp
from jax import lax
from jax.experimental import pallas as pl
from jax.experimental.pallas import tpu as pltpu
from jax.experimental.pallas import tpu_sc as plsc

TEMP = 0.2
LAMBDA_1 = 1e-4
EPS_N = 1e-12

_NC, _NSUB = 2, 16
_NW = _NC * _NSUB
_R = 1000


def _spmm4_sc(adj_row, adj_col, adj_val, x_es, x_ej, x_gs, x_gj):
    nj, d = x_es.shape
    e = adj_row.shape[0]
    ek = 80
    epw = e // _NW
    nblk = epw // ek
    slab = (nj // (8 * _NSUB)) * 8
    tail = nj - slab * _NSUB
    zr = slab // 4
    nv = d // 16

    mesh = plsc.VectorSubcoreMesh(core_axis_name="c", subcore_axis_name="s")

    def body(row_h, col_h, val_h, es_h, ej_h, gs_h, gj_h, out_h,
             accum, zbuf, idxs, idxd, vbuf, rows, sem):
        c = lax.axis_index("c")
        s = lax.axis_index("s")
        wid = s * _NC + c
        zero16 = jnp.zeros((16,), jnp.float32)

        def zb(i, carry):
            for j in range(nv):
                zbuf[i, pl.ds(j * 16, 16)] = zero16
            return carry
        lax.fori_loop(0, zr, zb, 0)

        slab0 = s * slab
        ebase0 = wid * epw
        for oi, (src_h, dst_h, x_h) in enumerate((
                (col_h, row_h, es_h), (row_h, col_h, ej_h),
                (col_h, row_h, gs_h), (row_h, col_h, gj_h))):
            for z in range(4):
                pltpu.sync_copy(zbuf, accum.at[pl.ds(slab0 + z * zr, zr)])

            @pl.when(s == _NSUB - 1)
            def _zero_tail():
                pltpu.sync_copy(zbuf.at[pl.ds(0, tail)],
                                accum.at[pl.ds(_NSUB * slab, tail)])
            plsc.subcore_barrier()

            def eblk(i, carry):
                base = ebase0 + i * ek
                pltpu.sync_copy(src_h.at[pl.ds(base, ek)], idxs)
                pltpu.sync_copy(dst_h.at[pl.ds(base, ek)], idxd)
                pltpu.sync_copy(val_h.at[pl.ds(base, ek)], vbuf)
                pltpu.async_copy(x_h.at[idxs], rows, sem).wait()

                def scale(g, carry2):
                    vv = vbuf[pl.ds(g * 16, 16)]
                    for t in range(16):
                        v = vv[t]
                        r = g * 16 + t
                        for j in range(nv):
                            sl = pl.ds(j * 16, 16)
                            rows[r, sl] = rows[r, sl] * v
                    return carry2
                lax.fori_loop(0, ek // 16, scale, 0)
                pltpu.sync_copy(rows, accum.at[idxd], add=True)
                return carry
            lax.fori_loop(0, nblk, eblk, 0)
            plsc.subcore_barrier()
            pltpu.sync_copy(accum.at[pl.ds(slab0, slab)],
                            out_h.at[oi].at[pl.ds(c * nj + slab0, slab)])

            @pl.when(s == _NSUB - 1)
            def _copy_tail():
                pltpu.sync_copy(
                    accum.at[pl.ds(_NSUB * slab, tail)],
                    out_h.at[oi].at[pl.ds(c * nj + _NSUB * slab, tail)])

    spmm4 = pl.kernel(
        body,
        out_type=jax.ShapeDtypeStruct((4, 2 * nj, d), jnp.float32),
        mesh=mesh,
        scratch_types=[
            pltpu.VMEM_SHARED((nj, d), jnp.float32),
            pltpu.VMEM((zr, d), jnp.float32),
            pltpu.VMEM((ek,), jnp.int32),
            pltpu.VMEM((ek,), jnp.int32),
            pltpu.VMEM((ek,), jnp.float32),
            pltpu.VMEM((ek, d), jnp.float32),
            pltpu.SemaphoreType.DMA,
        ],
    )
    return spmm4(adj_row, adj_col, adj_val, x_es, x_ej, x_gs, x_gj)


def _gather_sc(e_jn, e_sn, g_jn, g_sn, j_ids, s_ids, negflat):
    nj, d = e_jn.shape
    b = j_ids.shape[0]
    tn = negflat.shape[0]
    bpw = b // _NW
    npw = tn // _NW
    nk = npw // 128

    mesh = plsc.VectorSubcoreMesh(core_axis_name="c", subcore_axis_name="s")

    def body(ej_h, es_h, gj_h, gs_h, jid_h, sid_h, neg_h, osel_h, oneg_h,
             idxb, rowsb, idxn, rowsn, sem):
        c = lax.axis_index("c")
        s = lax.axis_index("s")
        wid = s * _NC + c
        base = wid * bpw
        pltpu.sync_copy(jid_h.at[pl.ds(base, bpw)], idxb)
        pltpu.async_copy(gj_h.at[idxb], rowsb, sem).wait()
        pltpu.sync_copy(rowsb, osel_h.at[0].at[pl.ds(base, bpw)])
        pltpu.async_copy(ej_h.at[idxb], rowsb, sem).wait()
        pltpu.sync_copy(rowsb, osel_h.at[1].at[pl.ds(base, bpw)])
        pltpu.sync_copy(sid_h.at[pl.ds(base, bpw)], idxb)
        pltpu.async_copy(gs_h.at[idxb], rowsb, sem).wait()
        pltpu.sync_copy(rowsb, osel_h.at[2].at[pl.ds(base, bpw)])
        pltpu.async_copy(es_h.at[idxb], rowsb, sem).wait()
        pltpu.sync_copy(rowsb, osel_h.at[3].at[pl.ds(base, bpw)])
        for k in range(nk):
            nb = wid * npw + k * 128
            pltpu.sync_copy(neg_h.at[pl.ds(nb, 128)], idxn)
            pltpu.async_copy(es_h.at[idxn], rowsn, sem).wait()
            pltpu.sync_copy(rowsn, oneg_h.at[pl.ds(nb, 128)])

    g = pl.kernel(
        body,
        out_type=(jax.ShapeDtypeStruct((4, b, d), jnp.float32),
                  jax.ShapeDtypeStruct((tn, d), jnp.float32)),
        mesh=mesh,
        scratch_types=[
            pltpu.VMEM((bpw,), jnp.int32),
            pltpu.VMEM((bpw, d), jnp.float32),
            pltpu.VMEM((128,), jnp.int32),
            pltpu.VMEM((128, d), jnp.float32),
            pltpu.SemaphoreType.DMA,
        ],
    )
    return g(e_jn, e_sn, g_jn, g_sn, j_ids, s_ids, negflat)


def _norm1_tc(x):
    n, d = x.shape

    def body(x_ref, o_ref):
        v = x_ref[...]
        nn = jnp.sqrt(jnp.sum(v * v, axis=1, keepdims=True))
        o_ref[...] = v / jnp.maximum(nn, EPS_N)

    return pl.pallas_call(
        body,
        grid=(n // _R,),
        in_specs=[pl.BlockSpec((_R, d), lambda i: (i, 0))],
        out_specs=pl.BlockSpec((_R, d), lambda i: (i, 0)),
        out_shape=jax.ShapeDtypeStruct((n, d), jnp.float32),
    )(x)


def _norm_mean_tc(xs):
    n, d = xs[0].shape
    k = len(xs)

    def body(*refs):
        o_ref = refs[-1]
        v = refs[0][...]
        for r in refs[1:-1]:
            v = v + r[...]
        v = v / float(k)
        nn = jnp.sqrt(jnp.sum(v * v, axis=1, keepdims=True))
        o_ref[...] = v / jnp.maximum(nn, EPS_N)

    return pl.pallas_call(
        body,
        grid=(n // _R,),
        in_specs=[pl.BlockSpec((_R, d), lambda i: (i, 0))] * k,
        out_specs=pl.BlockSpec((_R, d), lambda i: (i, 0)),
        out_shape=jax.ShapeDtypeStruct((n, d), jnp.float32),
    )(*xs)


def _layer_tc(p2, res, w, bvec):
    n, d = res.shape

    def body(p_ref, r_ref, w_ref, b_ref, o_ref):
        x = p_ref[0] + p_ref[1]
        y = lax.dot_general(x, w_ref[...], (((1,), (1,)), ((), ())),
                            preferred_element_type=jnp.float32)
        y = y + b_ref[...]
        o_ref[...] = r_ref[...] + jnp.where(y >= 0, y, y * 0.01)

    return pl.pallas_call(
        body,
        grid=(n // _R,),
        in_specs=[
            pl.BlockSpec((2, _R, d), lambda i: (0, i, 0)),
            pl.BlockSpec((_R, d), lambda i: (i, 0)),
            pl.BlockSpec((d, d), lambda i: (0, 0)),
            pl.BlockSpec((1, d), lambda i: (0, 0)),
        ],
        out_specs=pl.BlockSpec((_R, d), lambda i: (i, 0)),
        out_shape=jax.ShapeDtypeStruct((n, d), jnp.float32),
    )(p2, res, w, bvec.reshape(1, d))


def _score_tc(sel, neg_rows, wstk, bstk):
    nneg, b, d = neg_rows.shape

    def body(sel_ref, neg_ref, w_ref, bb_ref, o1, o2, op, org):
        i = pl.program_id(0)

        @pl.when(i == 0)
        def _():
            gj = sel_ref[0]
            ej = sel_ref[1]
            gs = sel_ref[2]
            es = sel_ref[3]
            s1 = lax.dot_general(gj, ej, (((1,), (1,)), ((), ())),
                                 preferred_element_type=jnp.float32)
            t1 = jnp.sum(jnp.exp(s1 / TEMP), axis=1)
            o1[...] = jnp.sum(jnp.log(t1 + 1e-8)).reshape(1, 1)
            pj = jnp.clip(jnp.sum(gj * ej, axis=1) / TEMP, -1.0, 1.0)
            ps = jnp.clip(jnp.sum(gs * es, axis=1) / TEMP, -1.0, 1.0)
            op[...] = (jnp.sum(pj) + jnp.sum(ps)).reshape(1, 1)
            org[...] = (jnp.sum(w_ref[...] * w_ref[...]) +
                        jnp.sum(bb_ref[...] * bb_ref[...])).reshape(1, 1)
            o2[...] = jnp.zeros((1, 1), jnp.float32)

        @pl.when(i > 0)
        def _():
            gs = sel_ref[2]
            sn = lax.dot_general(gs, neg_ref[0], (((1,), (1,)), ((), ())),
                                 preferred_element_type=jnp.float32)
            tn = jnp.sum(jnp.exp(sn / TEMP), axis=1)
            o2[...] = o2[...] + jnp.sum(jnp.log(tn + 1e-8)).reshape(1, 1)

    return pl.pallas_call(
        body,
        grid=(nneg + 1,),
        in_specs=[
            pl.BlockSpec((4, b, d), lambda i: (0, 0, 0)),
            pl.BlockSpec((1, b, d), lambda i: (jnp.maximum(i - 1, 0), 0, 0)),
            pl.BlockSpec(wstk.shape, lambda i: (0, 0, 0)),
            pl.BlockSpec(bstk.shape, lambda i: (0, 0)),
        ],
        out_specs=[pl.BlockSpec((1, 1), lambda i: (0, 0))] * 4,
        out_shape=[jax.ShapeDtypeStruct((1, 1), jnp.float32)] * 4,
    )(sel, neg_rows, wstk, bstk)


def kernel(e_j, e_s, g_j0, g_s0, adj_val, Wj, bj, Ws, bs, Wja, bja, Wsa, bsa,
           adj_row, adj_col, j_ids, s_ids, negs):
    nl = Wj.shape[0]
    nj, d = e_j.shape
    b = j_ids.shape[0]
    nneg = negs.shape[0]

    gj0 = _norm1_tc(g_j0)
    gs0 = _norm1_tc(g_s0)
    Ejs, Ess, Gjs, Gss = [e_j], [e_s], [gj0], [gs0]
    for l in range(nl):
        part = _spmm4_sc(adj_row, adj_col, adj_val,
                         Ess[l], Ejs[l], Gss[l], Gjs[l])
        part = part.reshape(4, 2, nj, d)
        Ejs.append(_layer_tc(part[0], Ejs[l], Wj[l], bj[l]))
        Ess.append(_layer_tc(part[1], Ess[l], Ws[l], bs[l]))
        Gjs.append(_layer_tc(part[2], Gjs[l], Wja[l], bja[l]))
        Gss.append(_layer_tc(part[3], Gss[l], Wsa[l], bsa[l]))

    e_jn = _norm_mean_tc(Ejs)
    e_sn = _norm_mean_tc(Ess)
    g_jn = _norm_mean_tc(Gjs)
    g_sn = _norm_mean_tc(Gss)

    sel, neg_rows = _gather_sc(e_jn, e_sn, g_jn, g_sn,
                               j_ids, s_ids, negs.reshape(-1))
    neg_rows = neg_rows.reshape(nneg, b, d)
    wstk = jnp.stack([Wj, Ws, Wja, Wsa]).reshape(-1, d, d)
    bstk = jnp.stack([bj, bs, bja, bsa]).reshape(-1, d)
    o1, o2, op, org = _score_tc(sel, neg_rows, wstk, bstk)

    neg_score = o1[0, 0] / b + o2[0, 0] / (b * nneg)
    pos_score = op[0, 0] / b
    loss_cl = (-pos_score + neg_score) * 0.2
    loss_reg = org[0, 0] * LAMBDA_1
    loss = loss_cl + loss_reg
    return (loss, loss_cl, loss_reg)


# speedup vs baseline: 6.9040x; 2.3540x over previous
import jax
import jax.numpy as jnp
from jax import lax
from jax.experimental import pallas as pl
from jax.experimental.pallas import tpu as pltpu
from jax.experimental.pallas import tpu_sc as plsc

TEMP = 0.2
LAMBDA_1 = 1e-4
EPS_N = 1e-12

_NC, _NSUB = 2, 16
_NW = _NC * _NSUB
_R = 1000


def _spmm4_sc(adj_row, adj_col, adj_val, x_es, x_ej, x_gs, x_gj):
    nj, d = x_es.shape
    e = adj_row.shape[0]
    ek = 80
    epw = e // _NW
    nblk = epw // ek
    npair = (nblk - 1) // 2
    slab = (nj // (8 * _NSUB)) * 8
    tail = nj - slab * _NSUB
    nzfull = slab // ek
    zrem = slab - nzfull * ek
    nv = d // 16

    mesh = plsc.VectorSubcoreMesh(core_axis_name="c", subcore_axis_name="s")

    def body(row_h, col_h, val_h, es_h, ej_h, gs_h, gj_h, out_h,
             accum, eidx_row, eidx_col, evals,
             didx0, didx1, rows0, rows1, gsem0, gsem1, ssem0, ssem1):
        c = lax.axis_index("c")
        s = lax.axis_index("s")
        wid = s * _NC + c
        zero16 = jnp.zeros((16,), jnp.float32)

        def zero_rows0():
            def zb(i, carry):
                for j in range(nv):
                    rows0[i, pl.ds(j * 16, 16)] = zero16
                return carry
            lax.fori_loop(0, ek, zb, 0)

        slab0 = s * slab
        ebase0 = wid * epw
        pltpu.sync_copy(row_h.at[pl.ds(ebase0, epw)], eidx_row)
        pltpu.sync_copy(col_h.at[pl.ds(ebase0, epw)], eidx_col)
        pltpu.sync_copy(val_h.at[pl.ds(ebase0, epw)], evals)

        didx = (didx0, didx1)
        rows = (rows0, rows1)
        gsem = (gsem0, gsem1)
        ssem = (ssem0, ssem1)

        def scale_rows(rb, i):
            def scale(g, carry2):
                vv = evals[pl.ds(i * ek + g * 16, 16)]
                for t in range(16):
                    v = vv[t]
                    for j in range(nv):
                        sl = pl.ds(j * 16, 16)
                        rb[g * 16 + t, sl] = rb[g * 16 + t, sl] * v
                return carry2
            lax.fori_loop(0, ek // 16, scale, 0)

        def fill_didx(b, src_big, i):
            for j in range(ek // 16):
                didx[b][pl.ds(j * 16, 16)] = src_big[pl.ds(i * ek + j * 16, 16)]

        for oi, (src_big, dst_big, x_h) in enumerate((
                (eidx_col, eidx_row, es_h), (eidx_row, eidx_col, ej_h),
                (eidx_col, eidx_row, gs_h), (eidx_row, eidx_col, gj_h))):
            zero_rows0()
            for z in range(nzfull):
                pltpu.sync_copy(rows0, accum.at[pl.ds(slab0 + z * ek, ek)])
            pltpu.sync_copy(rows0.at[pl.ds(0, zrem)],
                            accum.at[pl.ds(slab0 + nzfull * ek, zrem)])

            @pl.when(s == _NSUB - 1)
            def _zero_tail():
                pltpu.sync_copy(rows0.at[pl.ds(0, tail)],
                                accum.at[pl.ds(_NSUB * slab, tail)])
            plsc.subcore_barrier()

            def gather_start(b, i):
                pltpu.async_copy(
                    x_h.at[src_big.at[pl.ds(i * ek, ek)]], rows[b], gsem[b])

            def gather_wait(b):
                pltpu.make_async_copy(
                    x_h.at[src_big.at[pl.ds(0, ek)]], rows[b], gsem[b]).wait()

            def scatter_start(b):
                pltpu.async_copy(rows[b], accum.at[didx[b]], ssem[b], add=True)

            def scatter_wait(b):
                pltpu.make_async_copy(
                    rows[b], accum.at[didx[b]], ssem[b]).wait()

            gather_start(0, 0)

            def pair(g, carry):
                for b in range(2):
                    i = 2 * g + b
                    gather_wait(b)
                    if b == 0:
                        @pl.when(g >= 1)
                        def _():
                            scatter_wait(1)
                    else:
                        scatter_wait(0)
                    gather_start(1 - b, i + 1)
                    scale_rows(rows[b], i)
                    fill_didx(b, dst_big, i)
                    scatter_start(b)
                return carry
            lax.fori_loop(0, npair, pair, 0)

            gather_wait(0)
            scatter_wait(1)
            scale_rows(rows0, nblk - 1)
            fill_didx(0, dst_big, nblk - 1)
            pltpu.sync_copy(rows0, accum.at[didx0], add=True)

            plsc.subcore_barrier()
            pltpu.sync_copy(accum.at[pl.ds(slab0, slab)],
                            out_h.at[oi].at[pl.ds(c * nj + slab0, slab)])

            @pl.when(s == _NSUB - 1)
            def _copy_tail():
                pltpu.sync_copy(
                    accum.at[pl.ds(_NSUB * slab, tail)],
                    out_h.at[oi].at[pl.ds(c * nj + _NSUB * slab, tail)])

    spmm4 = pl.kernel(
        body,
        out_type=jax.ShapeDtypeStruct((4, 2 * nj, d), jnp.float32),
        mesh=mesh,
        scratch_types=[
            pltpu.VMEM_SHARED((nj, d), jnp.float32),
            pltpu.VMEM((epw,), jnp.int32),
            pltpu.VMEM((epw,), jnp.int32),
            pltpu.VMEM((epw,), jnp.float32),
            pltpu.VMEM((ek,), jnp.int32),
            pltpu.VMEM((ek,), jnp.int32),
            pltpu.VMEM((ek, d), jnp.float32),
            pltpu.VMEM((ek, d), jnp.float32),
            pltpu.SemaphoreType.DMA,
            pltpu.SemaphoreType.DMA,
            pltpu.SemaphoreType.DMA,
            pltpu.SemaphoreType.DMA,
        ],
    )
    return spmm4(adj_row, adj_col, adj_val, x_es, x_ej, x_gs, x_gj)


def _gather_sc(e_jn, e_sn, g_jn, g_sn, j_ids, s_ids, negflat):
    nj, d = e_jn.shape
    b = j_ids.shape[0]
    tn = negflat.shape[0]
    bpw = b // _NW
    npw = tn // _NW
    nk = npw // 128

    mesh = plsc.VectorSubcoreMesh(core_axis_name="c", subcore_axis_name="s")

    def body(ej_h, es_h, gj_h, gs_h, jid_h, sid_h, neg_h, osel_h, oneg_h,
             idxb, rowsb, idxn, rowsn, sem):
        c = lax.axis_index("c")
        s = lax.axis_index("s")
        wid = s * _NC + c
        base = wid * bpw
        pltpu.sync_copy(jid_h.at[pl.ds(base, bpw)], idxb)
        pltpu.async_copy(gj_h.at[idxb], rowsb, sem).wait()
        pltpu.sync_copy(rowsb, osel_h.at[0].at[pl.ds(base, bpw)])
        pltpu.async_copy(ej_h.at[idxb], rowsb, sem).wait()
        pltpu.sync_copy(rowsb, osel_h.at[1].at[pl.ds(base, bpw)])
        pltpu.sync_copy(sid_h.at[pl.ds(base, bpw)], idxb)
        pltpu.async_copy(gs_h.at[idxb], rowsb, sem).wait()
        pltpu.sync_copy(rowsb, osel_h.at[2].at[pl.ds(base, bpw)])
        pltpu.async_copy(es_h.at[idxb], rowsb, sem).wait()
        pltpu.sync_copy(rowsb, osel_h.at[3].at[pl.ds(base, bpw)])
        for k in range(nk):
            nb = wid * npw + k * 128
            pltpu.sync_copy(neg_h.at[pl.ds(nb, 128)], idxn)
            pltpu.async_copy(es_h.at[idxn], rowsn, sem).wait()
            pltpu.sync_copy(rowsn, oneg_h.at[pl.ds(nb, 128)])

    g = pl.kernel(
        body,
        out_type=(jax.ShapeDtypeStruct((4, b, d), jnp.float32),
                  jax.ShapeDtypeStruct((tn, d), jnp.float32)),
        mesh=mesh,
        scratch_types=[
            pltpu.VMEM((bpw,), jnp.int32),
            pltpu.VMEM((bpw, d), jnp.float32),
            pltpu.VMEM((128,), jnp.int32),
            pltpu.VMEM((128, d), jnp.float32),
            pltpu.SemaphoreType.DMA,
        ],
    )
    return g(e_jn, e_sn, g_jn, g_sn, j_ids, s_ids, negflat)


def _norm1_tc(x):
    n, d = x.shape

    def body(x_ref, o_ref):
        v = x_ref[...]
        nn = jnp.sqrt(jnp.sum(v * v, axis=1, keepdims=True))
        o_ref[...] = v / jnp.maximum(nn, EPS_N)

    return pl.pallas_call(
        body,
        grid=(n // _R,),
        in_specs=[pl.BlockSpec((_R, d), lambda i: (i, 0))],
        out_specs=pl.BlockSpec((_R, d), lambda i: (i, 0)),
        out_shape=jax.ShapeDtypeStruct((n, d), jnp.float32),
    )(x)


def _norm_mean_tc(xs):
    n, d = xs[0].shape
    k = len(xs)

    def body(*refs):
        o_ref = refs[-1]
        v = refs[0][...]
        for r in refs[1:-1]:
            v = v + r[...]
        v = v / float(k)
        nn = jnp.sqrt(jnp.sum(v * v, axis=1, keepdims=True))
        o_ref[...] = v / jnp.maximum(nn, EPS_N)

    return pl.pallas_call(
        body,
        grid=(n // _R,),
        in_specs=[pl.BlockSpec((_R, d), lambda i: (i, 0))] * k,
        out_specs=pl.BlockSpec((_R, d), lambda i: (i, 0)),
        out_shape=jax.ShapeDtypeStruct((n, d), jnp.float32),
    )(*xs)


def _layer_tc(p2, res, w, bvec):
    n, d = res.shape

    def body(p_ref, r_ref, w_ref, b_ref, o_ref):
        x = p_ref[0] + p_ref[1]
        y = lax.dot_general(x, w_ref[...], (((1,), (1,)), ((), ())),
                            preferred_element_type=jnp.float32)
        y = y + b_ref[...]
        o_ref[...] = r_ref[...] + jnp.where(y >= 0, y, y * 0.01)

    return pl.pallas_call(
        body,
        grid=(n // _R,),
        in_specs=[
            pl.BlockSpec((2, _R, d), lambda i: (0, i, 0)),
            pl.BlockSpec((_R, d), lambda i: (i, 0)),
            pl.BlockSpec((d, d), lambda i: (0, 0)),
            pl.BlockSpec((1, d), lambda i: (0, 0)),
        ],
        out_specs=pl.BlockSpec((_R, d), lambda i: (i, 0)),
        out_shape=jax.ShapeDtypeStruct((n, d), jnp.float32),
    )(p2, res, w, bvec.reshape(1, d))


def _score_tc(sel, neg_rows, wstk, bstk):
    nneg, b, d = neg_rows.shape

    def body(sel_ref, neg_ref, w_ref, bb_ref, o1, o2, op, org):
        i = pl.program_id(0)

        @pl.when(i == 0)
        def _():
            gj = sel_ref[0]
            ej = sel_ref[1]
            gs = sel_ref[2]
            es = sel_ref[3]
            s1 = lax.dot_general(gj, ej, (((1,), (1,)), ((), ())),
                                 preferred_element_type=jnp.float32)
            t1 = jnp.sum(jnp.exp(s1 / TEMP), axis=1)
            o1[...] = jnp.sum(jnp.log(t1 + 1e-8)).reshape(1, 1)
            pj = jnp.clip(jnp.sum(gj * ej, axis=1) / TEMP, -1.0, 1.0)
            ps = jnp.clip(jnp.sum(gs * es, axis=1) / TEMP, -1.0, 1.0)
            op[...] = (jnp.sum(pj) + jnp.sum(ps)).reshape(1, 1)
            org[...] = (jnp.sum(w_ref[...] * w_ref[...]) +
                        jnp.sum(bb_ref[...] * bb_ref[...])).reshape(1, 1)
            o2[...] = jnp.zeros((1, 1), jnp.float32)

        @pl.when(i > 0)
        def _():
            gs = sel_ref[2]
            sn = lax.dot_general(gs, neg_ref[0], (((1,), (1,)), ((), ())),
                                 preferred_element_type=jnp.float32)
            tn = jnp.sum(jnp.exp(sn / TEMP), axis=1)
            o2[...] = o2[...] + jnp.sum(jnp.log(tn + 1e-8)).reshape(1, 1)

    return pl.pallas_call(
        body,
        grid=(nneg + 1,),
        in_specs=[
            pl.BlockSpec((4, b, d), lambda i: (0, 0, 0)),
            pl.BlockSpec((1, b, d), lambda i: (jnp.maximum(i - 1, 0), 0, 0)),
            pl.BlockSpec(wstk.shape, lambda i: (0, 0, 0)),
            pl.BlockSpec(bstk.shape, lambda i: (0, 0)),
        ],
        out_specs=[pl.BlockSpec((1, 1), lambda i: (0, 0))] * 4,
        out_shape=[jax.ShapeDtypeStruct((1, 1), jnp.float32)] * 4,
    )(sel, neg_rows, wstk, bstk)


def kernel(e_j, e_s, g_j0, g_s0, adj_val, Wj, bj, Ws, bs, Wja, bja, Wsa, bsa,
           adj_row, adj_col, j_ids, s_ids, negs):
    nl = Wj.shape[0]
    nj, d = e_j.shape
    b = j_ids.shape[0]
    nneg = negs.shape[0]

    gj0 = _norm1_tc(g_j0)
    gs0 = _norm1_tc(g_s0)
    Ejs, Ess, Gjs, Gss = [e_j], [e_s], [gj0], [gs0]
    for l in range(nl):
        part = _spmm4_sc(adj_row, adj_col, adj_val,
                         Ess[l], Ejs[l], Gss[l], Gjs[l])
        part = part.reshape(4, 2, nj, d)
        Ejs.append(_layer_tc(part[0], Ejs[l], Wj[l], bj[l]))
        Ess.append(_layer_tc(part[1], Ess[l], Ws[l], bs[l]))
        Gjs.append(_layer_tc(part[2], Gjs[l], Wja[l], bja[l]))
        Gss.append(_layer_tc(part[3], Gss[l], Wsa[l], bsa[l]))

    e_jn = _norm_mean_tc(Ejs)
    e_sn = _norm_mean_tc(Ess)
    g_jn = _norm_mean_tc(Gjs)
    g_sn = _norm_mean_tc(Gss)

    sel, neg_rows = _gather_sc(e_jn, e_sn, g_jn, g_sn,
                               j_ids, s_ids, negs.reshape(-1))
    neg_rows = neg_rows.reshape(nneg, b, d)
    wstk = jnp.stack([Wj, Ws, Wja, Wsa]).reshape(-1, d, d)
    bstk = jnp.stack([bj, bs, bja, bsa]).reshape(-1, d)
    o1, o2, op, org = _score_tc(sel, neg_rows, wstk, bstk)

    neg_score = o1[0, 0] / b + o2[0, 0] / (b * nneg)
    pos_score = op[0, 0] / b
    loss_cl = (-pos_score + neg_score) * 0.2
    loss_reg = org[0, 0] * LAMBDA_1
    loss = loss_cl + loss_reg
    return (loss, loss_cl, loss_reg)
